# baseline (speedup 1.0000x reference)
import functools

import jax
import jax.numpy as jnp
from jax import lax
from jax.experimental import pallas as pl
from jax.experimental.pallas import tpu as pltpu
from jax.experimental.pallas import tpu_sc as plsc

N = 10000
NP = 10240
E = 320000
H = 128
DE = 16
NC, NS, L = 2, 16, 16
NW = NC * NS
CH = 128
KCH = 80
EPW = KCH * CH
EP = NW * EPW
RT = 512
NROWS_W = NP // NS

_mesh = lambda: plsc.VectorSubcoreMesh(core_axis_name="c", subcore_axis_name="s")


def _sc_deg_body(dstp, ewp, zeros_nb, out,
                 didx0, didx1, ewv0, ewv1, colbuf0, colbuf1, ssem0, ssem1, degS):
    c = lax.axis_index("c")
    s = lax.axis_index("s")
    wid = s * NC + c
    base = wid * EPW
    pltpu.sync_copy(zeros_nb, degS.at[pl.ds(s * NROWS_W, NROWS_W)])
    plsc.subcore_barrier()
    bufs = ((didx0, ewv0, colbuf0, ssem0), (didx1, ewv1, colbuf1, ssem1))

    def load(k, bi):
        didx, ewv, _, _ = bufs[bi]
        off = base + k * CH
        pltpu.sync_copy(dstp.at[pl.ds(off, CH)], didx)
        pltpu.sync_copy(ewp.at[pl.ds(off, CH)], ewv)

    def process(bi):
        didx, ewv, colbuf, ssem = bufs[bi]

        @plsc.parallel_loop(0, CH // 16, unroll=2)
        def _(t):
            ews = ewv[pl.ds(t * 16, 16)]
            for i in range(16):
                b = jnp.broadcast_to(ews[i], (16,))
                for j in range(H // 16):
                    colbuf[t * 16 + i, pl.ds(j * 16, 16)] = b

        return pltpu.async_copy(colbuf, degS.at[didx], ssem, add=True)

    load(0, 0)
    load(1, 1)

    def pair(k2, carry):
        k = 2 * k2
        cp0 = process(0)
        cp1 = process(1)
        cp0.wait()

        @pl.when(k2 < KCH // 2 - 1)
        def _():
            load(k + 2, 0)

        cp1.wait()

        @pl.when(k2 < KCH // 2 - 1)
        def _():
            load(k + 3, 1)

        return carry

    lax.fori_loop(0, KCH // 2, pair, 0)
    plsc.subcore_barrier()
    pltpu.sync_copy(degS.at[pl.ds(s * NROWS_W, NROWS_W)],
                    out.at[c, pl.ds(s * NROWS_W, NROWS_W)])


def _sc_deg(dstp, ewp, zeros_nb):
    k = pl.kernel(
        _sc_deg_body,
        out_type=jax.ShapeDtypeStruct((NC, NP, H), jnp.float32),
        mesh=_mesh(),
        compiler_params=pltpu.CompilerParams(needs_layout_passes=False),
        scratch_types=[
            pltpu.VMEM((CH,), jnp.int32),
            pltpu.VMEM((CH,), jnp.int32),
            pltpu.VMEM((CH,), jnp.float32),
            pltpu.VMEM((CH,), jnp.float32),
            pltpu.VMEM((CH, H), jnp.float32),
            pltpu.VMEM((CH, H), jnp.float32),
            pltpu.SemaphoreType.DMA,
            pltpu.SemaphoreType.DMA,
            pltpu.VMEM_SHARED((NP, H), jnp.float32),
        ],
    )
    return k(dstp, ewp, zeros_nb)


def _sc_gcn_body(hp, srcp, dstp, ewp, zeros_nb, out,
                 sidx0, sidx1, didx0, didx1, ewv0, ewv1, rows0, rows1,
                 gsem0, gsem1, ssem0, ssem1, aggS):
    c = lax.axis_index("c")
    s = lax.axis_index("s")
    wid = s * NC + c
    base = wid * EPW
    pltpu.sync_copy(zeros_nb, aggS.at[pl.ds(s * NROWS_W, NROWS_W)])
    plsc.subcore_barrier()
    bufs = ((sidx0, didx0, ewv0, rows0, gsem0, ssem0),
            (sidx1, didx1, ewv1, rows1, gsem1, ssem1))

    def load_and_gather(k, bi):
        sidx, didx, ewv, rows, gsem, _ = bufs[bi]
        off = base + k * CH
        pltpu.sync_copy(srcp.at[pl.ds(off, CH)], sidx)
        pltpu.async_copy(hp.at[sidx], rows, gsem)
        pltpu.sync_copy(dstp.at[pl.ds(off, CH)], didx)
        pltpu.sync_copy(ewp.at[pl.ds(off, CH)], ewv)

    def process(bi):
        sidx, didx, ewv, rows, gsem, ssem = bufs[bi]
        pltpu.make_async_copy(hp.at[sidx], rows, gsem).wait()

        @plsc.parallel_loop(0, CH // 16, unroll=2)
        def _(t):
            ews = ewv[pl.ds(t * 16, 16)]
            for i in range(16):
                sv = ews[i]
                e = t * 16 + i
                for j in range(H // 16):
                    rows[e, pl.ds(j * 16, 16)] = rows[e, pl.ds(j * 16, 16)] * sv

        return pltpu.async_copy(rows, aggS.at[didx], ssem, add=True)

    load_and_gather(0, 0)
    load_and_gather(1, 1)

    def pair(k2, carry):
        k = 2 * k2
        cp0 = process(0)
        cp1 = process(1)
        cp0.wait()

        @pl.when(k2 < KCH // 2 - 1)
        def _():
            load_and_gather(k + 2, 0)

        cp1.wait()

        @pl.when(k2 < KCH // 2 - 1)
        def _():
            load_and_gather(k + 3, 1)

        return carry

    lax.fori_loop(0, KCH // 2, pair, 0)
    plsc.subcore_barrier()
    pltpu.sync_copy(aggS.at[pl.ds(s * NROWS_W, NROWS_W)],
                    out.at[c, pl.ds(s * NROWS_W, NROWS_W)])


def _sc_gcn(hp, srcp, dstp, ewp, zeros_nb):
    k = pl.kernel(
        _sc_gcn_body,
        out_type=jax.ShapeDtypeStruct((NC, NP, H), jnp.float32),
        mesh=_mesh(),
        compiler_params=pltpu.CompilerParams(needs_layout_passes=False),
        scratch_types=[
            pltpu.VMEM((CH,), jnp.int32),
            pltpu.VMEM((CH,), jnp.int32),
            pltpu.VMEM((CH,), jnp.int32),
            pltpu.VMEM((CH,), jnp.int32),
            pltpu.VMEM((CH,), jnp.float32),
            pltpu.VMEM((CH,), jnp.float32),
            pltpu.VMEM((CH, H), jnp.float32),
            pltpu.VMEM((CH, H), jnp.float32),
            pltpu.SemaphoreType.DMA,
            pltpu.SemaphoreType.DMA,
            pltpu.SemaphoreType.DMA,
            pltpu.SemaphoreType.DMA,
            pltpu.VMEM_SHARED((NP, H), jnp.float32),
        ],
    )
    return k(hp, srcp, dstp, ewp, zeros_nb)


def _sc_dec_body(hA, hB, eab, srcp, dstp, w2, b2v, out,
                 aidx0, aidx1, bidx0, bidx1, bufA0, bufA1, bufB0, bufB1,
                 bufE0, bufE1, outv, w2l, b2l,
                 semA0, semA1, semB0, semB1, semE0, semE1):
    c = lax.axis_index("c")
    s = lax.axis_index("s")
    wid = s * NC + c
    base = wid * EPW
    pltpu.sync_copy(w2, w2l)
    pltpu.sync_copy(b2v, b2l)
    w2r = [w2l[pl.ds(j * 16, 16)] for j in range(H // 16)]
    b2r = b2l[...]
    iota16 = lax.iota(jnp.int32, 16)
    bufs = ((aidx0, bidx0, bufA0, bufB0, bufE0, semA0, semB0, semE0),
            (aidx1, bidx1, bufA1, bufB1, bufE1, semA1, semB1, semE1))

    base8 = wid * (EPW // PK)

    def load_and_gather(k, bi):
        aidx, bidx, bufA, bufB, bufE, semA, semB, semE = bufs[bi]
        off = base + k * CH
        off8 = base8 + k * (CH // PK)
        pltpu.sync_copy(srcp.at[pl.ds(off, CH)], aidx)
        pltpu.sync_copy(dstp.at[pl.ds(off, CH)], bidx)
        pltpu.async_copy(hA.at[aidx], bufA, semA)
        pltpu.async_copy(hB.at[bidx], bufB, semB)
        pltpu.async_copy(eab.at[pl.ds(off8, CH // PK)], bufE, semE)

    def process(k, bi):
        aidx, bidx, bufA, bufB, bufE, semA, semB, semE = bufs[bi]
        off = base + k * CH
        off8 = base8 + k * (CH // PK)
        pltpu.make_async_copy(hA.at[aidx], bufA, semA).wait()
        pltpu.make_async_copy(hB.at[bidx], bufB, semB).wait()
        pltpu.make_async_copy(eab.at[pl.ds(off8, CH // PK)], bufE, semE).wait()

        @plsc.parallel_loop(0, CH // 16, unroll=2)
        def _(t):
            o = b2r
            for i in range(16):
                e = t * 16 + i
                er = 2 * t + i // PK
                cb = (i % PK) * H
                acc = jnp.zeros((16,), jnp.float32)
                for j in range(H // 16):
                    v = (bufA[e, pl.ds(j * 16, 16)] + bufB[e, pl.ds(j * 16, 16)]
                         + bufE[er, pl.ds(cb + j * 16, 16)])
                    v = jnp.maximum(v, 0.0)
                    acc = acc + v * w2r[j]
                o = jnp.where(iota16 == i, jnp.sum(acc), o)
            outv[pl.ds(t * 16, 16)] = o

        pltpu.sync_copy(outv, out.at[pl.ds(off, CH)])

    load_and_gather(0, 0)
    load_and_gather(1, 1)

    def pair(k2, carry):
        k = 2 * k2
        process(k, 0)

        @pl.when(k2 < KCH // 2 - 1)
        def _():
            load_and_gather(k + 2, 0)

        process(k + 1, 1)

        @pl.when(k2 < KCH // 2 - 1)
        def _():
            load_and_gather(k + 3, 1)

        return carry

    lax.fori_loop(0, KCH // 2, pair, 0)


def _sc_dec(hA, hB, eab, srcp, dstp, w2, b2v):
    k = pl.kernel(
        _sc_dec_body,
        out_type=jax.ShapeDtypeStruct((EP,), jnp.float32),
        mesh=_mesh(),
        compiler_params=pltpu.CompilerParams(needs_layout_passes=False),
        scratch_types=[
            pltpu.VMEM((CH,), jnp.int32),
            pltpu.VMEM((CH,), jnp.int32),
            pltpu.VMEM((CH,), jnp.int32),
            pltpu.VMEM((CH,), jnp.int32),
            pltpu.VMEM((CH, H), jnp.float32),
            pltpu.VMEM((CH, H), jnp.float32),
            pltpu.VMEM((CH, H), jnp.float32),
            pltpu.VMEM((CH, H), jnp.float32),
            pltpu.VMEM((CH // PK, PK * H), jnp.float32),
            pltpu.VMEM((CH // PK, PK * H), jnp.float32),
            pltpu.VMEM((CH,), jnp.float32),
            pltpu.VMEM((H,), jnp.float32),
            pltpu.VMEM((16,), jnp.float32),
            pltpu.SemaphoreType.DMA,
            pltpu.SemaphoreType.DMA,
            pltpu.SemaphoreType.DMA,
            pltpu.SemaphoreType.DMA,
            pltpu.SemaphoreType.DMA,
            pltpu.SemaphoreType.DMA,
        ],
    )
    return k(hA, hB, eab, srcp, dstp, w2, b2v)


def _tc_prep_body(x_ref, wx_ref, bx_ref, degp_ref, hp0_ref, dinv_ref):
    d = degp_ref[0][:, 0:1] + degp_ref[1][:, 0:1]
    dinv = jnp.where(d > 0, lax.rsqrt(jnp.maximum(d, 1e-12)), 0.0)
    h = jnp.dot(x_ref[...], wx_ref[...], preferred_element_type=jnp.float32) + bx_ref[...]
    hp0_ref[...] = h * dinv
    dinv_ref[...] = dinv


def _tc_prep(x_pad, Wx, bx2, degp):
    grid = (NP // RT,)
    return pl.pallas_call(
        _tc_prep_body,
        grid=grid,
        in_specs=[
            pl.BlockSpec((RT, H), lambda i: (i, 0)),
            pl.BlockSpec((H, H), lambda i: (0, 0)),
            pl.BlockSpec((1, H), lambda i: (0, 0)),
            pl.BlockSpec((NC, RT, H), lambda i: (0, i, 0)),
        ],
        out_specs=[
            pl.BlockSpec((RT, H), lambda i: (i, 0)),
            pl.BlockSpec((RT, 1), lambda i: (i, 0)),
        ],
        out_shape=[
            jax.ShapeDtypeStruct((NP, H), jnp.float32),
            jax.ShapeDtypeStruct((NP, 1), jnp.float32),
        ],
    )(x_pad, Wx, bx2, degp)


def _tc_layer1_body(ragg_ref, dinv_ref, w_ref, b_ref, hp_ref):
    dv = dinv_ref[...]
    agg = (ragg_ref[0] + ragg_ref[1]) * dv
    h = jnp.maximum(jnp.dot(agg, w_ref[...], preferred_element_type=jnp.float32)
                    + b_ref[...], 0.0)
    hp_ref[...] = h * dv


def _tc_layer1(ragg, dinv, W, b2):
    grid = (NP // RT,)
    return pl.pallas_call(
        _tc_layer1_body,
        grid=grid,
        in_specs=[
            pl.BlockSpec((NC, RT, H), lambda i: (0, i, 0)),
            pl.BlockSpec((RT, 1), lambda i: (i, 0)),
            pl.BlockSpec((H, H), lambda i: (0, 0)),
            pl.BlockSpec((1, H), lambda i: (0, 0)),
        ],
        out_specs=pl.BlockSpec((RT, H), lambda i: (i, 0)),
        out_shape=jax.ShapeDtypeStruct((NP, H), jnp.float32),
    )(ragg, dinv, W, b2)


def _tc_layer2_body(ragg_ref, dinv_ref, w_ref, b_ref, a_ref, bb_ref, hA_ref, hB_ref):
    dv = dinv_ref[...]
    agg = (ragg_ref[0] + ragg_ref[1]) * dv
    t = jnp.maximum(jnp.dot(agg, w_ref[...], preferred_element_type=jnp.float32)
                    + b_ref[...], 0.0)
    hA_ref[...] = jnp.dot(t, a_ref[...], preferred_element_type=jnp.float32)
    hB_ref[...] = jnp.dot(t, bb_ref[...], preferred_element_type=jnp.float32)


def _tc_layer2(ragg, dinv, W, b2, A, B):
    grid = (NP // RT,)
    return pl.pallas_call(
        _tc_layer2_body,
        grid=grid,
        in_specs=[
            pl.BlockSpec((NC, RT, H), lambda i: (0, i, 0)),
            pl.BlockSpec((RT, 1), lambda i: (i, 0)),
            pl.BlockSpec((H, H), lambda i: (0, 0)),
            pl.BlockSpec((1, H), lambda i: (0, 0)),
            pl.BlockSpec((H, H), lambda i: (0, 0)),
            pl.BlockSpec((H, H), lambda i: (0, 0)),
        ],
        out_specs=[
            pl.BlockSpec((RT, H), lambda i: (i, 0)),
            pl.BlockSpec((RT, H), lambda i: (i, 0)),
        ],
        out_shape=[
            jax.ShapeDtypeStruct((NP, H), jnp.float32),
            jax.ShapeDtypeStruct((NP, H), jnp.float32),
        ],
    )(ragg, dinv, W, b2, A, B)


PK = 8
EAR = EP // PK
EABT = 64


def _tc_ea_body(ea_ref, wbd_ref, cvec_ref, out_ref):
    out_ref[...] = (jnp.dot(ea_ref[...], wbd_ref[...], preferred_element_type=jnp.float32)
                    + cvec_ref[...])


def _tc_ea(eap8, Wbd, cvec8):
    grid = (EAR // EABT,)
    return pl.pallas_call(
        _tc_ea_body,
        grid=grid,
        in_specs=[
            pl.BlockSpec((EABT, H), lambda i: (i, 0)),
            pl.BlockSpec((H, PK * H), lambda i: (0, 0)),
            pl.BlockSpec((1, PK * H), lambda i: (0, 0)),
        ],
        out_specs=pl.BlockSpec((EABT, PK * H), lambda i: (i, 0)),
        out_shape=jax.ShapeDtypeStruct((EAR, PK * H), jnp.float32),
    )(eap8, Wbd, cvec8)


def kernel(x, edge_index, edge_attr, edge_weight, Wx, bx, We, be,
           Wg0, bg0, Wg1, bg1, Wd1, bd1, Wd2, bd2):
    src = edge_index[0]
    dst = edge_index[1]

    x_pad = jnp.pad(x, ((0, NP - N), (0, 0)))
    srcp = jnp.pad(src, (0, EP - E))
    dstp = jnp.pad(dst, (0, EP - E))
    ewp = jnp.pad(edge_weight, (0, EP - E))
    eap8 = jnp.pad(edge_attr.reshape(E // PK, PK * DE), ((0, EAR - E // PK), (0, 0)))

    A = Wd1[:H]
    B = Wd1[H:2 * H]
    C = Wd1[2 * H:]
    Wec = We @ C
    cvec = (be @ C + bd1).reshape(1, H)
    Wbd = jnp.zeros((PK * DE, PK * H), jnp.float32)
    for r in range(PK):
        Wbd = Wbd.at[r * DE:(r + 1) * DE, r * H:(r + 1) * H].set(Wec)
    cvec8 = jnp.tile(cvec, (1, PK))
    bx2 = bx.reshape(1, H)
    bg02 = bg0.reshape(1, H)
    bg12 = bg1.reshape(1, H)
    w2 = Wd2.reshape(H)
    b2v = jnp.full((16,), bd2[0], jnp.float32)

    zeros_agg = jnp.zeros((NROWS_W, H), jnp.float32)

    degp = _sc_deg(dstp, ewp, zeros_agg)
    hp0, dinv = _tc_prep(x_pad, Wx, bx2, degp)

    ragg1 = _sc_gcn(hp0, srcp, dstp, ewp, zeros_agg)
    hp1 = _tc_layer1(ragg1, dinv, Wg0, bg02)
    ragg2 = _sc_gcn(hp1, srcp, dstp, ewp, zeros_agg)
    hA, hB = _tc_layer2(ragg2, dinv, Wg1, bg12, A, B)

    eab = _tc_ea(eap8, Wbd, cvec8)
    dec = _sc_dec(hA, hB, eab, srcp, dstp, w2, b2v)
    return dec[:E].reshape(E, 1)


# speedup vs baseline: 1.2309x; 1.2309x over previous
import functools

import jax
import jax.numpy as jnp
from jax import lax
from jax.experimental import pallas as pl
from jax.experimental.pallas import tpu as pltpu
from jax.experimental.pallas import tpu_sc as plsc

N = 10000
NP = 10240
E = 320000
H = 128
DE = 16
NC, NS, L = 2, 16, 16
NW = NC * NS
CH = 128
KCH = 80
EPW = KCH * CH
EP = NW * EPW
RT = 512
NROWS_W = NP // NS

_mesh = lambda: plsc.VectorSubcoreMesh(core_axis_name="c", subcore_axis_name="s")


def _sc_deg_body(dstp, ewp, zeros_nb, out,
                 didx0, didx1, ewv0, ewv1, colbuf0, colbuf1, ssem0, ssem1, degS):
    c = lax.axis_index("c")
    s = lax.axis_index("s")
    wid = s * NC + c
    base = wid * EPW
    pltpu.sync_copy(zeros_nb, degS.at[pl.ds(s * NROWS_W, NROWS_W)])
    plsc.subcore_barrier()
    bufs = ((didx0, ewv0, colbuf0, ssem0), (didx1, ewv1, colbuf1, ssem1))

    def load(k, bi):
        didx, ewv, _, _ = bufs[bi]
        off = base + k * CH
        pltpu.sync_copy(dstp.at[pl.ds(off, CH)], didx)
        pltpu.sync_copy(ewp.at[pl.ds(off, CH)], ewv)

    def process(bi):
        didx, ewv, colbuf, ssem = bufs[bi]

        @plsc.parallel_loop(0, CH // 16, unroll=2)
        def _(t):
            ews = ewv[pl.ds(t * 16, 16)]
            for i in range(16):
                b = jnp.broadcast_to(ews[i], (16,))
                for j in range(H // 16):
                    colbuf[t * 16 + i, pl.ds(j * 16, 16)] = b

        return pltpu.async_copy(colbuf, degS.at[didx], ssem, add=True)

    load(0, 0)
    load(1, 1)

    def pair(k2, carry):
        k = 2 * k2
        cp0 = process(0)
        cp1 = process(1)
        cp0.wait()

        @pl.when(k2 < KCH // 2 - 1)
        def _():
            load(k + 2, 0)

        cp1.wait()

        @pl.when(k2 < KCH // 2 - 1)
        def _():
            load(k + 3, 1)

        return carry

    lax.fori_loop(0, KCH // 2, pair, 0)
    plsc.subcore_barrier()
    pltpu.sync_copy(degS.at[pl.ds(s * NROWS_W, NROWS_W)],
                    out.at[c, pl.ds(s * NROWS_W, NROWS_W)])


def _sc_deg(dstp, ewp, zeros_nb):
    k = pl.kernel(
        _sc_deg_body,
        out_type=jax.ShapeDtypeStruct((NC, NP, H), jnp.float32),
        mesh=_mesh(),
        compiler_params=pltpu.CompilerParams(needs_layout_passes=False),
        scratch_types=[
            pltpu.VMEM((CH,), jnp.int32),
            pltpu.VMEM((CH,), jnp.int32),
            pltpu.VMEM((CH,), jnp.float32),
            pltpu.VMEM((CH,), jnp.float32),
            pltpu.VMEM((CH, H), jnp.float32),
            pltpu.VMEM((CH, H), jnp.float32),
            pltpu.SemaphoreType.DMA,
            pltpu.SemaphoreType.DMA,
            pltpu.VMEM_SHARED((NP, H), jnp.float32),
        ],
    )
    return k(dstp, ewp, zeros_nb)


def _sc_gcn_body(hp, srcp, dstp, ewp, zeros_nb, out,
                 sidx0, sidx1, didx0, didx1, ewv0, ewv1, rows0, rows1,
                 gsem0, gsem1, ssem0, ssem1, aggS):
    c = lax.axis_index("c")
    s = lax.axis_index("s")
    wid = s * NC + c
    base = wid * EPW
    pltpu.sync_copy(zeros_nb, aggS.at[pl.ds(s * NROWS_W, NROWS_W)])
    plsc.subcore_barrier()
    bufs = ((sidx0, didx0, ewv0, rows0, gsem0, ssem0),
            (sidx1, didx1, ewv1, rows1, gsem1, ssem1))

    def load_and_gather(k, bi):
        sidx, didx, ewv, rows, gsem, _ = bufs[bi]
        off = base + k * CH
        pltpu.sync_copy(srcp.at[pl.ds(off, CH)], sidx)
        pltpu.async_copy(hp.at[sidx], rows, gsem)
        pltpu.sync_copy(dstp.at[pl.ds(off, CH)], didx)
        pltpu.sync_copy(ewp.at[pl.ds(off, CH)], ewv)

    def process(bi):
        sidx, didx, ewv, rows, gsem, ssem = bufs[bi]
        pltpu.make_async_copy(hp.at[sidx], rows, gsem).wait()

        @plsc.parallel_loop(0, CH // 16, unroll=2)
        def _(t):
            ews = ewv[pl.ds(t * 16, 16)]
            for i in range(16):
                sv = ews[i]
                e = t * 16 + i
                for j in range(H // 16):
                    rows[e, pl.ds(j * 16, 16)] = rows[e, pl.ds(j * 16, 16)] * sv

        return pltpu.async_copy(rows, aggS.at[didx], ssem, add=True)

    load_and_gather(0, 0)
    load_and_gather(1, 1)

    def pair(k2, carry):
        k = 2 * k2
        cp0 = process(0)
        cp1 = process(1)
        cp0.wait()

        @pl.when(k2 < KCH // 2 - 1)
        def _():
            load_and_gather(k + 2, 0)

        cp1.wait()

        @pl.when(k2 < KCH // 2 - 1)
        def _():
            load_and_gather(k + 3, 1)

        return carry

    lax.fori_loop(0, KCH // 2, pair, 0)
    plsc.subcore_barrier()
    pltpu.sync_copy(aggS.at[pl.ds(s * NROWS_W, NROWS_W)],
                    out.at[c, pl.ds(s * NROWS_W, NROWS_W)])


def _sc_gcn(hp, srcp, dstp, ewp, zeros_nb):
    k = pl.kernel(
        _sc_gcn_body,
        out_type=jax.ShapeDtypeStruct((NC, NP, H), jnp.float32),
        mesh=_mesh(),
        compiler_params=pltpu.CompilerParams(needs_layout_passes=False),
        scratch_types=[
            pltpu.VMEM((CH,), jnp.int32),
            pltpu.VMEM((CH,), jnp.int32),
            pltpu.VMEM((CH,), jnp.int32),
            pltpu.VMEM((CH,), jnp.int32),
            pltpu.VMEM((CH,), jnp.float32),
            pltpu.VMEM((CH,), jnp.float32),
            pltpu.VMEM((CH, H), jnp.float32),
            pltpu.VMEM((CH, H), jnp.float32),
            pltpu.SemaphoreType.DMA,
            pltpu.SemaphoreType.DMA,
            pltpu.SemaphoreType.DMA,
            pltpu.SemaphoreType.DMA,
            pltpu.VMEM_SHARED((NP, H), jnp.float32),
        ],
    )
    return k(hp, srcp, dstp, ewp, zeros_nb)


def _sc_dec_body(hA, hB, eab, srcp, dstp, w2, b2v, out,
                 aidx0, aidx1, bidx0, bidx1, bufA0, bufA1, bufB0, bufB1,
                 bufE0, bufE1, outv, w2l, b2l,
                 semA0, semA1, semB0, semB1, semE0, semE1):
    c = lax.axis_index("c")
    s = lax.axis_index("s")
    wid = s * NC + c
    base = wid * EPW
    pltpu.sync_copy(w2, w2l)
    pltpu.sync_copy(b2v, b2l)
    w2r = [w2l[pl.ds(j * 16, 16)] for j in range(H // 16)]
    b2r = b2l[...]
    iota16 = lax.iota(jnp.int32, 16)
    bufs = ((aidx0, bidx0, bufA0, bufB0, bufE0, semA0, semB0, semE0),
            (aidx1, bidx1, bufA1, bufB1, bufE1, semA1, semB1, semE1))

    def load_and_gather(k, bi):
        aidx, bidx, bufA, bufB, bufE, semA, semB, semE = bufs[bi]
        off = base + k * CH
        pltpu.sync_copy(srcp.at[pl.ds(off, CH)], aidx)
        pltpu.sync_copy(dstp.at[pl.ds(off, CH)], bidx)
        pltpu.async_copy(hA.at[aidx], bufA, semA)
        pltpu.async_copy(hB.at[bidx], bufB, semB)
        pltpu.async_copy(eab.at[pl.ds(off, CH)], bufE, semE)

    def process(k, bi):
        aidx, bidx, bufA, bufB, bufE, semA, semB, semE = bufs[bi]
        off = base + k * CH
        pltpu.make_async_copy(hA.at[aidx], bufA, semA).wait()
        pltpu.make_async_copy(hB.at[bidx], bufB, semB).wait()
        pltpu.make_async_copy(eab.at[pl.ds(off, CH)], bufE, semE).wait()

        @plsc.parallel_loop(0, CH // 16, unroll=1)
        def _(t):
            o = b2r
            for i in range(16):
                e = t * 16 + i
                acc = jnp.zeros((16,), jnp.float32)
                for j in range(H // 16):
                    v = (bufA[e, pl.ds(j * 16, 16)] + bufB[e, pl.ds(j * 16, 16)]
                         + bufE[e, pl.ds(j * 16, 16)])
                    v = jnp.maximum(v, 0.0)
                    acc = acc + v * w2r[j]
                o = jnp.where(iota16 == i, jnp.sum(acc), o)
            outv[pl.ds(t * 16, 16)] = o

        pltpu.sync_copy(outv, out.at[pl.ds(off, CH)])

    load_and_gather(0, 0)
    load_and_gather(1, 1)

    def pair(k2, carry):
        k = 2 * k2
        process(k, 0)

        @pl.when(k2 < KCH // 2 - 1)
        def _():
            load_and_gather(k + 2, 0)

        process(k + 1, 1)

        @pl.when(k2 < KCH // 2 - 1)
        def _():
            load_and_gather(k + 3, 1)

        return carry

    lax.fori_loop(0, KCH // 2, pair, 0)


def _sc_dec(hA, hB, eab, srcp, dstp, w2, b2v):
    k = pl.kernel(
        _sc_dec_body,
        out_type=jax.ShapeDtypeStruct((EP,), jnp.float32),
        mesh=_mesh(),
        compiler_params=pltpu.CompilerParams(needs_layout_passes=False),
        scratch_types=[
            pltpu.VMEM((CH,), jnp.int32),
            pltpu.VMEM((CH,), jnp.int32),
            pltpu.VMEM((CH,), jnp.int32),
            pltpu.VMEM((CH,), jnp.int32),
            pltpu.VMEM((CH, H), jnp.float32),
            pltpu.VMEM((CH, H), jnp.float32),
            pltpu.VMEM((CH, H), jnp.float32),
            pltpu.VMEM((CH, H), jnp.float32),
            pltpu.VMEM((CH, H), jnp.float32),
            pltpu.VMEM((CH, H), jnp.float32),
            pltpu.VMEM((CH,), jnp.float32),
            pltpu.VMEM((H,), jnp.float32),
            pltpu.VMEM((16,), jnp.float32),
            pltpu.SemaphoreType.DMA,
            pltpu.SemaphoreType.DMA,
            pltpu.SemaphoreType.DMA,
            pltpu.SemaphoreType.DMA,
            pltpu.SemaphoreType.DMA,
            pltpu.SemaphoreType.DMA,
        ],
    )
    return k(hA, hB, eab, srcp, dstp, w2, b2v)


def _tc_prep_body(x_ref, wx_ref, bx_ref, degp_ref, hp0_ref, dinv_ref):
    d = degp_ref[0][:, 0:1] + degp_ref[1][:, 0:1]
    dinv = jnp.where(d > 0, lax.rsqrt(jnp.maximum(d, 1e-12)), 0.0)
    h = jnp.dot(x_ref[...], wx_ref[...], preferred_element_type=jnp.float32) + bx_ref[...]
    hp0_ref[...] = h * dinv
    dinv_ref[...] = dinv


def _tc_prep(x_pad, Wx, bx2, degp):
    grid = (NP // RT,)
    return pl.pallas_call(
        _tc_prep_body,
        grid=grid,
        in_specs=[
            pl.BlockSpec((RT, H), lambda i: (i, 0)),
            pl.BlockSpec((H, H), lambda i: (0, 0)),
            pl.BlockSpec((1, H), lambda i: (0, 0)),
            pl.BlockSpec((NC, RT, H), lambda i: (0, i, 0)),
        ],
        out_specs=[
            pl.BlockSpec((RT, H), lambda i: (i, 0)),
            pl.BlockSpec((RT, 1), lambda i: (i, 0)),
        ],
        out_shape=[
            jax.ShapeDtypeStruct((NP, H), jnp.float32),
            jax.ShapeDtypeStruct((NP, 1), jnp.float32),
        ],
    )(x_pad, Wx, bx2, degp)


def _tc_layer1_body(ragg_ref, dinv_ref, w_ref, b_ref, hp_ref):
    dv = dinv_ref[...]
    agg = (ragg_ref[0] + ragg_ref[1]) * dv
    h = jnp.maximum(jnp.dot(agg, w_ref[...], preferred_element_type=jnp.float32)
                    + b_ref[...], 0.0)
    hp_ref[...] = h * dv


def _tc_layer1(ragg, dinv, W, b2):
    grid = (NP // RT,)
    return pl.pallas_call(
        _tc_layer1_body,
        grid=grid,
        in_specs=[
            pl.BlockSpec((NC, RT, H), lambda i: (0, i, 0)),
            pl.BlockSpec((RT, 1), lambda i: (i, 0)),
            pl.BlockSpec((H, H), lambda i: (0, 0)),
            pl.BlockSpec((1, H), lambda i: (0, 0)),
        ],
        out_specs=pl.BlockSpec((RT, H), lambda i: (i, 0)),
        out_shape=jax.ShapeDtypeStruct((NP, H), jnp.float32),
    )(ragg, dinv, W, b2)


def _tc_layer2_body(ragg_ref, dinv_ref, w_ref, b_ref, a_ref, bb_ref, hA_ref, hB_ref):
    dv = dinv_ref[...]
    agg = (ragg_ref[0] + ragg_ref[1]) * dv
    t = jnp.maximum(jnp.dot(agg, w_ref[...], preferred_element_type=jnp.float32)
                    + b_ref[...], 0.0)
    hA_ref[...] = jnp.dot(t, a_ref[...], preferred_element_type=jnp.float32)
    hB_ref[...] = jnp.dot(t, bb_ref[...], preferred_element_type=jnp.float32)


def _tc_layer2(ragg, dinv, W, b2, A, B):
    grid = (NP // RT,)
    return pl.pallas_call(
        _tc_layer2_body,
        grid=grid,
        in_specs=[
            pl.BlockSpec((NC, RT, H), lambda i: (0, i, 0)),
            pl.BlockSpec((RT, 1), lambda i: (i, 0)),
            pl.BlockSpec((H, H), lambda i: (0, 0)),
            pl.BlockSpec((1, H), lambda i: (0, 0)),
            pl.BlockSpec((H, H), lambda i: (0, 0)),
            pl.BlockSpec((H, H), lambda i: (0, 0)),
        ],
        out_specs=[
            pl.BlockSpec((RT, H), lambda i: (i, 0)),
            pl.BlockSpec((RT, H), lambda i: (i, 0)),
        ],
        out_shape=[
            jax.ShapeDtypeStruct((NP, H), jnp.float32),
            jax.ShapeDtypeStruct((NP, H), jnp.float32),
        ],
    )(ragg, dinv, W, b2, A, B)


def _tc_ea_body(eat_ref, wec_ref, cvec_ref, out_ref):
    ea = lax.dot_general(eat_ref[...], wec_ref[...],
                         (((0,), (0,)), ((), ())),
                         preferred_element_type=jnp.float32)
    out_ref[...] = ea + cvec_ref[...]


def _tc_ea(attrT, Wec, cvec2):
    grid = (E // RT,)
    return pl.pallas_call(
        _tc_ea_body,
        grid=grid,
        in_specs=[
            pl.BlockSpec((DE, RT), lambda i: (0, i)),
            pl.BlockSpec((DE, H), lambda i: (0, 0)),
            pl.BlockSpec((1, H), lambda i: (0, 0)),
        ],
        out_specs=pl.BlockSpec((RT, H), lambda i: (i, 0)),
        out_shape=jax.ShapeDtypeStruct((EP, H), jnp.float32),
    )(attrT, Wec, cvec2)


def kernel(x, edge_index, edge_attr, edge_weight, Wx, bx, We, be,
           Wg0, bg0, Wg1, bg1, Wd1, bd1, Wd2, bd2):
    src = edge_index[0]
    dst = edge_index[1]

    x_pad = jnp.pad(x, ((0, NP - N), (0, 0)))
    srcp = jnp.pad(src, (0, EP - E))
    dstp = jnp.pad(dst, (0, EP - E))
    ewp = jnp.pad(edge_weight, (0, EP - E))
    attrT = edge_attr.T

    A = Wd1[:H]
    B = Wd1[H:2 * H]
    C = Wd1[2 * H:]
    Wec = We @ C
    cvec = (be @ C + bd1).reshape(1, H)
    bx2 = bx.reshape(1, H)
    bg02 = bg0.reshape(1, H)
    bg12 = bg1.reshape(1, H)
    w2 = Wd2.reshape(H)
    b2v = jnp.full((16,), bd2[0], jnp.float32)

    zeros_agg = jnp.zeros((NROWS_W, H), jnp.float32)

    degp = _sc_deg(dstp, ewp, zeros_agg)
    hp0, dinv = _tc_prep(x_pad, Wx, bx2, degp)

    ragg1 = _sc_gcn(hp0, srcp, dstp, ewp, zeros_agg)
    hp1 = _tc_layer1(ragg1, dinv, Wg0, bg02)
    ragg2 = _sc_gcn(hp1, srcp, dstp, ewp, zeros_agg)
    hA, hB = _tc_layer2(ragg2, dinv, Wg1, bg12, A, B)

    eab = _tc_ea(attrT, Wec, cvec)
    dec = _sc_dec(hA, hB, eab, srcp, dstp, w2, b2v)
    return dec[:E].reshape(E, 1)


# speedup vs baseline: 1.7608x; 1.4304x over previous
import functools

import jax
import jax.numpy as jnp
from jax import lax
from jax.experimental import pallas as pl
from jax.experimental.pallas import tpu as pltpu
from jax.experimental.pallas import tpu_sc as plsc

N = 10000
NP = 10240
E = 320000
H = 128
DE = 16
NC, NS, L = 2, 16, 16
NW = NC * NS
CH = 128
KCH = 80
EPW = KCH * CH
EP = NW * EPW
RT = 512
NROWS_W = NP // NS

_mesh = lambda: plsc.VectorSubcoreMesh(core_axis_name="c", subcore_axis_name="s")


def _sc_deg_body(dstp, ewp, zeros_nb, out,
                 didx0, didx1, ewv0, ewv1, colbuf0, colbuf1, ssem0, ssem1, degS):
    c = lax.axis_index("c")
    s = lax.axis_index("s")
    wid = s * NC + c
    base = wid * EPW
    pltpu.sync_copy(zeros_nb, degS.at[pl.ds(s * NROWS_W, NROWS_W)])
    plsc.subcore_barrier()
    bufs = ((didx0, ewv0, colbuf0, ssem0), (didx1, ewv1, colbuf1, ssem1))

    def load(k, bi):
        didx, ewv, _, _ = bufs[bi]
        off = base + k * CH
        pltpu.sync_copy(dstp.at[pl.ds(off, CH)], didx)
        pltpu.sync_copy(ewp.at[pl.ds(off, CH)], ewv)

    def process(bi):
        didx, ewv, colbuf, ssem = bufs[bi]

        @plsc.parallel_loop(0, CH // 16, unroll=2)
        def _(t):
            ews = ewv[pl.ds(t * 16, 16)]
            for i in range(16):
                b = jnp.broadcast_to(ews[i], (16,))
                for j in range(H // 16):
                    colbuf[t * 16 + i, pl.ds(j * 16, 16)] = b

        return pltpu.async_copy(colbuf, degS.at[didx], ssem, add=True)

    load(0, 0)
    load(1, 1)

    def pair(k2, carry):
        k = 2 * k2
        cp0 = process(0)
        cp1 = process(1)
        cp0.wait()

        @pl.when(k2 < KCH // 2 - 1)
        def _():
            load(k + 2, 0)

        cp1.wait()

        @pl.when(k2 < KCH // 2 - 1)
        def _():
            load(k + 3, 1)

        return carry

    lax.fori_loop(0, KCH // 2, pair, 0)
    plsc.subcore_barrier()
    pltpu.sync_copy(degS.at[pl.ds(s * NROWS_W, NROWS_W)],
                    out.at[c, pl.ds(s * NROWS_W, NROWS_W)])


def _sc_deg(dstp, ewp, zeros_nb):
    k = pl.kernel(
        _sc_deg_body,
        out_type=jax.ShapeDtypeStruct((NC, NP, H), jnp.float32),
        mesh=_mesh(),
        compiler_params=pltpu.CompilerParams(needs_layout_passes=False),
        scratch_types=[
            pltpu.VMEM((CH,), jnp.int32),
            pltpu.VMEM((CH,), jnp.int32),
            pltpu.VMEM((CH,), jnp.float32),
            pltpu.VMEM((CH,), jnp.float32),
            pltpu.VMEM((CH, H), jnp.float32),
            pltpu.VMEM((CH, H), jnp.float32),
            pltpu.SemaphoreType.DMA,
            pltpu.SemaphoreType.DMA,
            pltpu.VMEM_SHARED((NP, H), jnp.float32),
        ],
    )
    return k(dstp, ewp, zeros_nb)


def _sc_gcn_body(hp, srcp, dstp, ewp, zeros_nb, out,
                 sidx0, sidx1, didx0, didx1, ewv0, ewv1, rows0, rows1,
                 gsem0, gsem1, ssem0, ssem1, aggS):
    c = lax.axis_index("c")
    s = lax.axis_index("s")
    wid = s * NC + c
    base = wid * EPW
    pltpu.sync_copy(zeros_nb, aggS.at[pl.ds(s * NROWS_W, NROWS_W)])
    plsc.subcore_barrier()
    bufs = ((sidx0, didx0, ewv0, rows0, gsem0, ssem0),
            (sidx1, didx1, ewv1, rows1, gsem1, ssem1))

    def load_and_gather(k, bi):
        sidx, didx, ewv, rows, gsem, _ = bufs[bi]
        off = base + k * CH
        pltpu.sync_copy(srcp.at[pl.ds(off, CH)], sidx)
        pltpu.async_copy(hp.at[sidx], rows, gsem)
        pltpu.sync_copy(dstp.at[pl.ds(off, CH)], didx)
        pltpu.sync_copy(ewp.at[pl.ds(off, CH)], ewv)

    def process(bi):
        sidx, didx, ewv, rows, gsem, ssem = bufs[bi]
        pltpu.make_async_copy(hp.at[sidx], rows, gsem).wait()

        @plsc.parallel_loop(0, CH // 16, unroll=2)
        def _(t):
            ews = ewv[pl.ds(t * 16, 16)]
            for i in range(16):
                sv = ews[i]
                e = t * 16 + i
                for j in range(H // 16):
                    rows[e, pl.ds(j * 16, 16)] = rows[e, pl.ds(j * 16, 16)] * sv

        return pltpu.async_copy(rows, aggS.at[didx], ssem, add=True)

    load_and_gather(0, 0)
    load_and_gather(1, 1)

    def pair(k2, carry):
        k = 2 * k2
        cp0 = process(0)
        cp1 = process(1)
        cp0.wait()

        @pl.when(k2 < KCH // 2 - 1)
        def _():
            load_and_gather(k + 2, 0)

        cp1.wait()

        @pl.when(k2 < KCH // 2 - 1)
        def _():
            load_and_gather(k + 3, 1)

        return carry

    lax.fori_loop(0, KCH // 2, pair, 0)
    plsc.subcore_barrier()
    pltpu.sync_copy(aggS.at[pl.ds(s * NROWS_W, NROWS_W)],
                    out.at[c, pl.ds(s * NROWS_W, NROWS_W)])


def _sc_gcn(hp, srcp, dstp, ewp, zeros_nb):
    k = pl.kernel(
        _sc_gcn_body,
        out_type=jax.ShapeDtypeStruct((NC, NP, H), jnp.float32),
        mesh=_mesh(),
        compiler_params=pltpu.CompilerParams(needs_layout_passes=False),
        scratch_types=[
            pltpu.VMEM((CH,), jnp.int32),
            pltpu.VMEM((CH,), jnp.int32),
            pltpu.VMEM((CH,), jnp.int32),
            pltpu.VMEM((CH,), jnp.int32),
            pltpu.VMEM((CH,), jnp.float32),
            pltpu.VMEM((CH,), jnp.float32),
            pltpu.VMEM((CH, H), jnp.float32),
            pltpu.VMEM((CH, H), jnp.float32),
            pltpu.SemaphoreType.DMA,
            pltpu.SemaphoreType.DMA,
            pltpu.SemaphoreType.DMA,
            pltpu.SemaphoreType.DMA,
            pltpu.VMEM_SHARED((NP, H), jnp.float32),
        ],
    )
    return k(hp, srcp, dstp, ewp, zeros_nb)


def _sc_dec_body(hA, hB, eab, srcp, dstp, w2, b2v, out,
                 aidx0, aidx1, bidx0, bidx1, bufA0, bufA1, bufB0, bufB1,
                 bufE0, bufE1, outv, w2l, b2l,
                 semA0, semA1, semB0, semB1, semE0, semE1):
    c = lax.axis_index("c")
    s = lax.axis_index("s")
    wid = s * NC + c
    base = wid * EPW
    pltpu.sync_copy(w2, w2l)
    pltpu.sync_copy(b2v, b2l)
    w2r = [w2l[pl.ds(j * 16, 16)] for j in range(H // 16)]
    b2r = b2l[...]
    iota16 = lax.iota(jnp.int32, 16)
    bufs = ((aidx0, bidx0, bufA0, bufB0, bufE0, semA0, semB0, semE0),
            (aidx1, bidx1, bufA1, bufB1, bufE1, semA1, semB1, semE1))

    def load_and_gather(k, bi):
        aidx, bidx, bufA, bufB, bufE, semA, semB, semE = bufs[bi]
        off = base + k * CH
        pltpu.sync_copy(srcp.at[pl.ds(off, CH)], aidx)
        pltpu.sync_copy(dstp.at[pl.ds(off, CH)], bidx)
        pltpu.async_copy(hA.at[aidx], bufA, semA)
        pltpu.async_copy(hB.at[bidx], bufB, semB)
        pltpu.async_copy(eab.at[pl.ds(off, CH)], bufE, semE)

    def process(k, bi):
        aidx, bidx, bufA, bufB, bufE, semA, semB, semE = bufs[bi]
        off = base + k * CH
        pltpu.make_async_copy(hA.at[aidx], bufA, semA).wait()
        pltpu.make_async_copy(hB.at[bidx], bufB, semB).wait()
        pltpu.make_async_copy(eab.at[pl.ds(off, CH)], bufE, semE).wait()

        @plsc.parallel_loop(0, CH // 16, unroll=1)
        def _(t):
            o = b2r
            for i in range(16):
                e = t * 16 + i
                acc = jnp.zeros((16,), jnp.float32)
                for j in range(H // 16):
                    v = (bufA[e, pl.ds(j * 16, 16)] + bufB[e, pl.ds(j * 16, 16)]
                         + bufE[e, pl.ds(j * 16, 16)])
                    v = jnp.maximum(v, 0.0)
                    acc = acc + v * w2r[j]
                o = jnp.where(iota16 == i, jnp.sum(acc), o)
            outv[pl.ds(t * 16, 16)] = o

        pltpu.sync_copy(outv, out.at[pl.ds(off, CH)])

    load_and_gather(0, 0)
    load_and_gather(1, 1)

    def pair(k2, carry):
        k = 2 * k2
        process(k, 0)

        @pl.when(k2 < KCH // 2 - 1)
        def _():
            load_and_gather(k + 2, 0)

        process(k + 1, 1)

        @pl.when(k2 < KCH // 2 - 1)
        def _():
            load_and_gather(k + 3, 1)

        return carry

    lax.fori_loop(0, KCH // 2, pair, 0)


def _sc_dec(hA, hB, eab, srcp, dstp, w2, b2v):
    k = pl.kernel(
        _sc_dec_body,
        out_type=jax.ShapeDtypeStruct((EP,), jnp.float32),
        mesh=_mesh(),
        compiler_params=pltpu.CompilerParams(needs_layout_passes=False),
        scratch_types=[
            pltpu.VMEM((CH,), jnp.int32),
            pltpu.VMEM((CH,), jnp.int32),
            pltpu.VMEM((CH,), jnp.int32),
            pltpu.VMEM((CH,), jnp.int32),
            pltpu.VMEM((CH, H), jnp.float32),
            pltpu.VMEM((CH, H), jnp.float32),
            pltpu.VMEM((CH, H), jnp.float32),
            pltpu.VMEM((CH, H), jnp.float32),
            pltpu.VMEM((CH, H), jnp.float32),
            pltpu.VMEM((CH, H), jnp.float32),
            pltpu.VMEM((CH,), jnp.float32),
            pltpu.VMEM((H,), jnp.float32),
            pltpu.VMEM((16,), jnp.float32),
            pltpu.SemaphoreType.DMA,
            pltpu.SemaphoreType.DMA,
            pltpu.SemaphoreType.DMA,
            pltpu.SemaphoreType.DMA,
            pltpu.SemaphoreType.DMA,
            pltpu.SemaphoreType.DMA,
        ],
    )
    return k(hA, hB, eab, srcp, dstp, w2, b2v)


def _tc_prep_body(x_ref, wx_ref, bx_ref, degp_ref, hp0_ref, dinv_ref):
    d = degp_ref[0][:, 0:1] + degp_ref[1][:, 0:1]
    dinv = jnp.where(d > 0, lax.rsqrt(jnp.maximum(d, 1e-12)), 0.0)
    h = jnp.dot(x_ref[...], wx_ref[...], preferred_element_type=jnp.float32) + bx_ref[...]
    hp0_ref[...] = h * dinv
    dinv_ref[...] = dinv


def _tc_prep(x_pad, Wx, bx2, degp):
    grid = (NP // RT,)
    return pl.pallas_call(
        _tc_prep_body,
        grid=grid,
        in_specs=[
            pl.BlockSpec((RT, H), lambda i: (i, 0)),
            pl.BlockSpec((H, H), lambda i: (0, 0)),
            pl.BlockSpec((1, H), lambda i: (0, 0)),
            pl.BlockSpec((NC, RT, H), lambda i: (0, i, 0)),
        ],
        out_specs=[
            pl.BlockSpec((RT, H), lambda i: (i, 0)),
            pl.BlockSpec((RT, 1), lambda i: (i, 0)),
        ],
        out_shape=[
            jax.ShapeDtypeStruct((NP, H), jnp.float32),
            jax.ShapeDtypeStruct((NP, 1), jnp.float32),
        ],
    )(x_pad, Wx, bx2, degp)


def _tc_layer1_body(ragg_ref, dinv_ref, w_ref, b_ref, hp_ref):
    dv = dinv_ref[...]
    agg = (ragg_ref[0] + ragg_ref[1]) * dv
    h = jnp.maximum(jnp.dot(agg, w_ref[...], preferred_element_type=jnp.float32)
                    + b_ref[...], 0.0)
    hp_ref[...] = h * dv


def _tc_layer1(ragg, dinv, W, b2):
    grid = (NP // RT,)
    return pl.pallas_call(
        _tc_layer1_body,
        grid=grid,
        in_specs=[
            pl.BlockSpec((NC, RT, H), lambda i: (0, i, 0)),
            pl.BlockSpec((RT, 1), lambda i: (i, 0)),
            pl.BlockSpec((H, H), lambda i: (0, 0)),
            pl.BlockSpec((1, H), lambda i: (0, 0)),
        ],
        out_specs=pl.BlockSpec((RT, H), lambda i: (i, 0)),
        out_shape=jax.ShapeDtypeStruct((NP, H), jnp.float32),
    )(ragg, dinv, W, b2)


def _tc_layer2_body(ragg_ref, dinv_ref, w_ref, b_ref, a_ref, bb_ref, hA_ref, hB_ref):
    dv = dinv_ref[...]
    agg = (ragg_ref[0] + ragg_ref[1]) * dv
    t = jnp.maximum(jnp.dot(agg, w_ref[...], preferred_element_type=jnp.float32)
                    + b_ref[...], 0.0)
    hA_ref[...] = jnp.dot(t, a_ref[...], preferred_element_type=jnp.float32)
    hB_ref[...] = jnp.dot(t, bb_ref[...], preferred_element_type=jnp.float32)


def _tc_layer2(ragg, dinv, W, b2, A, B):
    grid = (NP // RT,)
    return pl.pallas_call(
        _tc_layer2_body,
        grid=grid,
        in_specs=[
            pl.BlockSpec((NC, RT, H), lambda i: (0, i, 0)),
            pl.BlockSpec((RT, 1), lambda i: (i, 0)),
            pl.BlockSpec((H, H), lambda i: (0, 0)),
            pl.BlockSpec((1, H), lambda i: (0, 0)),
            pl.BlockSpec((H, H), lambda i: (0, 0)),
            pl.BlockSpec((H, H), lambda i: (0, 0)),
        ],
        out_specs=[
            pl.BlockSpec((RT, H), lambda i: (i, 0)),
            pl.BlockSpec((RT, H), lambda i: (i, 0)),
        ],
        out_shape=[
            jax.ShapeDtypeStruct((NP, H), jnp.float32),
            jax.ShapeDtypeStruct((NP, H), jnp.float32),
        ],
    )(ragg, dinv, W, b2, A, B)


def _tc_ea_body(eat_ref, wec_ref, cvec_ref, out_ref):
    ea = lax.dot_general(eat_ref[...], wec_ref[...],
                         (((0,), (0,)), ((), ())),
                         preferred_element_type=jnp.float32)
    out_ref[...] = ea + cvec_ref[...]


def _tc_ea(attrT, Wec, cvec2):
    grid = (E // RT,)
    return pl.pallas_call(
        _tc_ea_body,
        grid=grid,
        in_specs=[
            pl.BlockSpec((DE, RT), lambda i: (0, i)),
            pl.BlockSpec((DE, H), lambda i: (0, 0)),
            pl.BlockSpec((1, H), lambda i: (0, 0)),
        ],
        out_specs=pl.BlockSpec((RT, H), lambda i: (i, 0)),
        out_shape=jax.ShapeDtypeStruct((EP, H), jnp.float32),
    )(attrT, Wec, cvec2)


def kernel(x, edge_index, edge_attr, edge_weight, Wx, bx, We, be,
           Wg0, bg0, Wg1, bg1, Wd1, bd1, Wd2, bd2):
    src = edge_index[0]
    dst = edge_index[1]

    x_pad = jnp.pad(x, ((0, NP - N), (0, 0)))
    pad_idx = (jnp.arange(EP - E, dtype=jnp.int32) % (NP - N)) + N
    srcp = jnp.concatenate([src, pad_idx])
    dstp = jnp.concatenate([dst, pad_idx])
    ewp = jnp.pad(edge_weight, (0, EP - E))
    attrT = edge_attr.T

    A = Wd1[:H]
    B = Wd1[H:2 * H]
    C = Wd1[2 * H:]
    Wec = We @ C
    cvec = (be @ C + bd1).reshape(1, H)
    bx2 = bx.reshape(1, H)
    bg02 = bg0.reshape(1, H)
    bg12 = bg1.reshape(1, H)
    w2 = Wd2.reshape(H)
    b2v = jnp.full((16,), bd2[0], jnp.float32)

    zeros_agg = jnp.zeros((NROWS_W, H), jnp.float32)

    degp = _sc_deg(dstp, ewp, zeros_agg)
    hp0, dinv = _tc_prep(x_pad, Wx, bx2, degp)

    ragg1 = _sc_gcn(hp0, srcp, dstp, ewp, zeros_agg)
    hp1 = _tc_layer1(ragg1, dinv, Wg0, bg02)
    ragg2 = _sc_gcn(hp1, srcp, dstp, ewp, zeros_agg)
    hA, hB = _tc_layer2(ragg2, dinv, Wg1, bg12, A, B)

    eab = _tc_ea(attrT, Wec, cvec)
    dec = _sc_dec(hA, hB, eab, srcp, dstp, w2, b2v)
    return dec[:E].reshape(E, 1)


# speedup vs baseline: 1.8183x; 1.0327x over previous
import functools

import jax
import jax.numpy as jnp
from jax import lax
from jax.experimental import pallas as pl
from jax.experimental.pallas import tpu as pltpu
from jax.experimental.pallas import tpu_sc as plsc

N = 10000
NP = 10240
E = 320000
H = 128
DE = 16
NC, NS, L = 2, 16, 16
NW = NC * NS
CH = 128
KCH = 80
EPW = KCH * CH
EP = NW * EPW
RT = 512
NROWS_W = NP // NS

_mesh = lambda: plsc.VectorSubcoreMesh(core_axis_name="c", subcore_axis_name="s")


def _sc_deg_body(dstp, ewp, zeros_nb, out,
                 didx0, didx1, ewv0, ewv1, colbuf0, colbuf1, ssem0, ssem1, degS):
    c = lax.axis_index("c")
    s = lax.axis_index("s")
    wid = s * NC + c
    base = wid * EPW
    pltpu.sync_copy(zeros_nb, degS.at[pl.ds(s * NROWS_W, NROWS_W)])
    plsc.subcore_barrier()
    bufs = ((didx0, ewv0, colbuf0, ssem0), (didx1, ewv1, colbuf1, ssem1))

    def load(k, bi):
        didx, ewv, _, _ = bufs[bi]
        off = base + k * CH
        pltpu.sync_copy(dstp.at[pl.ds(off, CH)], didx)
        pltpu.sync_copy(ewp.at[pl.ds(off, CH)], ewv)

    def process(bi):
        didx, ewv, colbuf, ssem = bufs[bi]

        @plsc.parallel_loop(0, CH // 16, unroll=2)
        def _(t):
            ews = ewv[pl.ds(t * 16, 16)]
            for i in range(16):
                b = jnp.broadcast_to(ews[i], (16,))
                for j in range(H // 16):
                    colbuf[t * 16 + i, pl.ds(j * 16, 16)] = b

        return pltpu.async_copy(colbuf, degS.at[didx], ssem, add=True)

    load(0, 0)
    load(1, 1)

    def pair(k2, carry):
        k = 2 * k2
        cp0 = process(0)
        cp1 = process(1)
        cp0.wait()

        @pl.when(k2 < KCH // 2 - 1)
        def _():
            load(k + 2, 0)

        cp1.wait()

        @pl.when(k2 < KCH // 2 - 1)
        def _():
            load(k + 3, 1)

        return carry

    lax.fori_loop(0, KCH // 2, pair, 0)
    plsc.subcore_barrier()
    pltpu.sync_copy(degS.at[pl.ds(s * NROWS_W, NROWS_W)],
                    out.at[c, pl.ds(s * NROWS_W, NROWS_W)])


def _sc_deg(dstp, ewp, zeros_nb):
    k = pl.kernel(
        _sc_deg_body,
        out_type=jax.ShapeDtypeStruct((NC, NP, H), jnp.float32),
        mesh=_mesh(),
        compiler_params=pltpu.CompilerParams(needs_layout_passes=False),
        scratch_types=[
            pltpu.VMEM((CH,), jnp.int32),
            pltpu.VMEM((CH,), jnp.int32),
            pltpu.VMEM((CH,), jnp.float32),
            pltpu.VMEM((CH,), jnp.float32),
            pltpu.VMEM((CH, H), jnp.float32),
            pltpu.VMEM((CH, H), jnp.float32),
            pltpu.SemaphoreType.DMA,
            pltpu.SemaphoreType.DMA,
            pltpu.VMEM_SHARED((NP, H), jnp.float32),
        ],
    )
    return k(dstp, ewp, zeros_nb)


def _sc_gcn_body(hp, srcp, dstp, ewp, zeros_nb, out,
                 sidx0, sidx1, didx0, didx1, ewv0, ewv1, rows0, rows1,
                 gsem0, gsem1, ssem0, ssem1, aggS):
    c = lax.axis_index("c")
    s = lax.axis_index("s")
    wid = s * NC + c
    base = wid * EPW
    pltpu.sync_copy(zeros_nb, aggS.at[pl.ds(s * NROWS_W, NROWS_W)])
    plsc.subcore_barrier()
    bufs = ((sidx0, didx0, ewv0, rows0, gsem0, ssem0),
            (sidx1, didx1, ewv1, rows1, gsem1, ssem1))

    def load_and_gather(k, bi):
        sidx, didx, ewv, rows, gsem, _ = bufs[bi]
        off = base + k * CH
        pltpu.sync_copy(srcp.at[pl.ds(off, CH)], sidx)
        pltpu.async_copy(hp.at[sidx], rows, gsem)
        pltpu.sync_copy(dstp.at[pl.ds(off, CH)], didx)
        pltpu.sync_copy(ewp.at[pl.ds(off, CH)], ewv)

    def process(bi):
        sidx, didx, ewv, rows, gsem, ssem = bufs[bi]
        pltpu.make_async_copy(hp.at[sidx], rows, gsem).wait()

        @plsc.parallel_loop(0, CH // 16, unroll=2)
        def _(t):
            ews = ewv[pl.ds(t * 16, 16)]
            for i in range(16):
                sv = ews[i]
                e = t * 16 + i
                for j in range(H // 16):
                    rows[e, pl.ds(j * 16, 16)] = rows[e, pl.ds(j * 16, 16)] * sv

        return pltpu.async_copy(rows, aggS.at[didx], ssem, add=True)

    load_and_gather(0, 0)
    load_and_gather(1, 1)

    def pair(k2, carry):
        k = 2 * k2
        cp0 = process(0)
        cp1 = process(1)
        cp0.wait()

        @pl.when(k2 < KCH // 2 - 1)
        def _():
            load_and_gather(k + 2, 0)

        cp1.wait()

        @pl.when(k2 < KCH // 2 - 1)
        def _():
            load_and_gather(k + 3, 1)

        return carry

    lax.fori_loop(0, KCH // 2, pair, 0)
    plsc.subcore_barrier()
    pltpu.sync_copy(aggS.at[pl.ds(s * NROWS_W, NROWS_W)],
                    out.at[c, pl.ds(s * NROWS_W, NROWS_W)])


def _sc_gcn(hp, srcp, dstp, ewp, zeros_nb):
    k = pl.kernel(
        _sc_gcn_body,
        out_type=jax.ShapeDtypeStruct((NC, NP, H), jnp.float32),
        mesh=_mesh(),
        compiler_params=pltpu.CompilerParams(needs_layout_passes=False),
        scratch_types=[
            pltpu.VMEM((CH,), jnp.int32),
            pltpu.VMEM((CH,), jnp.int32),
            pltpu.VMEM((CH,), jnp.int32),
            pltpu.VMEM((CH,), jnp.int32),
            pltpu.VMEM((CH,), jnp.float32),
            pltpu.VMEM((CH,), jnp.float32),
            pltpu.VMEM((CH, H), jnp.float32),
            pltpu.VMEM((CH, H), jnp.float32),
            pltpu.SemaphoreType.DMA,
            pltpu.SemaphoreType.DMA,
            pltpu.SemaphoreType.DMA,
            pltpu.SemaphoreType.DMA,
            pltpu.VMEM_SHARED((NP, H), jnp.float32),
        ],
    )
    return k(hp, srcp, dstp, ewp, zeros_nb)


EPH = EP // 2
EPWH = EPH // NW
KCHH = EPWH // CH


def _sc_gab_body(half, hA, hB, srcp, dstp, out,
                 aidx0, aidx1, bidx0, bidx1, bufA0, bufA1, bufB0, bufB1,
                 semA0, semA1, semB0, semB1, wsem0, wsem1):
    c = lax.axis_index("c")
    s = lax.axis_index("s")
    wid = s * NC + c
    base = half * EPH + wid * EPWH
    obase = wid * EPWH
    bufs = ((aidx0, bidx0, bufA0, bufB0, semA0, semB0, wsem0),
            (aidx1, bidx1, bufA1, bufB1, semA1, semB1, wsem1))

    def load_and_gather(k, bi):
        aidx, bidx, bufA, bufB, semA, semB, _ = bufs[bi]
        off = base + k * CH
        pltpu.sync_copy(srcp.at[pl.ds(off, CH)], aidx)
        pltpu.sync_copy(dstp.at[pl.ds(off, CH)], bidx)
        pltpu.async_copy(hA.at[aidx], bufA, semA)
        pltpu.async_copy(hB.at[bidx], bufB, semB)

    def process(k, bi):
        aidx, bidx, bufA, bufB, semA, semB, wsem = bufs[bi]
        ooff = obase + k * CH
        pltpu.make_async_copy(hA.at[aidx], bufA, semA).wait()
        pltpu.make_async_copy(hB.at[bidx], bufB, semB).wait()

        @plsc.parallel_loop(0, CH // 16, unroll=2)
        def _(t):
            for i in range(16):
                e = t * 16 + i
                for j in range(H // 16):
                    sl = pl.ds(j * 16, 16)
                    bufA[e, sl] = bufA[e, sl] + bufB[e, sl]

        return pltpu.async_copy(bufA, out.at[pl.ds(ooff, CH)], wsem)

    load_and_gather(0, 0)
    load_and_gather(1, 1)

    def pair(k2, carry):
        k = 2 * k2
        cp0 = process(k, 0)
        cp1 = process(k + 1, 1)
        cp0.wait()

        @pl.when(k2 < KCHH // 2 - 1)
        def _():
            load_and_gather(k + 2, 0)

        cp1.wait()

        @pl.when(k2 < KCHH // 2 - 1)
        def _():
            load_and_gather(k + 3, 1)

        return carry

    lax.fori_loop(0, KCHH // 2, pair, 0)


def _sc_gab(hA, hB, srcp, dstp, half):
    k = pl.kernel(
        functools.partial(_sc_gab_body, half),
        out_type=jax.ShapeDtypeStruct((EPH, H), jnp.float32),
        mesh=_mesh(),
        compiler_params=pltpu.CompilerParams(needs_layout_passes=False),
        scratch_types=[
            pltpu.VMEM((CH,), jnp.int32),
            pltpu.VMEM((CH,), jnp.int32),
            pltpu.VMEM((CH,), jnp.int32),
            pltpu.VMEM((CH,), jnp.int32),
            pltpu.VMEM((CH, H), jnp.float32),
            pltpu.VMEM((CH, H), jnp.float32),
            pltpu.VMEM((CH, H), jnp.float32),
            pltpu.VMEM((CH, H), jnp.float32),
            pltpu.SemaphoreType.DMA,
            pltpu.SemaphoreType.DMA,
            pltpu.SemaphoreType.DMA,
            pltpu.SemaphoreType.DMA,
            pltpu.SemaphoreType.DMA,
            pltpu.SemaphoreType.DMA,
        ],
    )
    return k(hA, hB, srcp, dstp)


def _tc_dec_body(gab_ref, attrT_ref, wec_ref, cvec_ref, w2_ref, b2_ref, out_ref):
    ea = lax.dot_general(attrT_ref[...], wec_ref[...],
                         (((0,), (0,)), ((), ())),
                         preferred_element_type=jnp.float32)
    t = jnp.maximum(gab_ref[...] + ea + cvec_ref[...], 0.0)
    out_ref[...] = jnp.sum(t * w2_ref[...], axis=1, keepdims=True) + b2_ref[...]


def _tc_dec(gab, attrT, Wec, cvec2, w2row, b2s, n_rows, attr_off):
    grid = (n_rows // RT,)
    return pl.pallas_call(
        _tc_dec_body,
        grid=grid,
        in_specs=[
            pl.BlockSpec((RT, H), lambda i: (i, 0)),
            pl.BlockSpec((DE, RT), lambda i: (0, i + attr_off)),
            pl.BlockSpec((DE, H), lambda i: (0, 0)),
            pl.BlockSpec((1, H), lambda i: (0, 0)),
            pl.BlockSpec((1, H), lambda i: (0, 0)),
            pl.BlockSpec((1, 1), lambda i: (0, 0)),
        ],
        out_specs=pl.BlockSpec((RT, 1), lambda i: (i, 0)),
        out_shape=jax.ShapeDtypeStruct((n_rows, 1), jnp.float32),
    )(gab, attrT, Wec, cvec2, w2row, b2s)


def _tc_prep_body(x_ref, wx_ref, bx_ref, degp_ref, hp0_ref, dinv_ref):
    d = degp_ref[0][:, 0:1] + degp_ref[1][:, 0:1]
    dinv = jnp.where(d > 0, lax.rsqrt(jnp.maximum(d, 1e-12)), 0.0)
    h = jnp.dot(x_ref[...], wx_ref[...], preferred_element_type=jnp.float32) + bx_ref[...]
    hp0_ref[...] = h * dinv
    dinv_ref[...] = dinv


def _tc_prep(x_pad, Wx, bx2, degp):
    grid = (NP // RT,)
    return pl.pallas_call(
        _tc_prep_body,
        grid=grid,
        in_specs=[
            pl.BlockSpec((RT, H), lambda i: (i, 0)),
            pl.BlockSpec((H, H), lambda i: (0, 0)),
            pl.BlockSpec((1, H), lambda i: (0, 0)),
            pl.BlockSpec((NC, RT, H), lambda i: (0, i, 0)),
        ],
        out_specs=[
            pl.BlockSpec((RT, H), lambda i: (i, 0)),
            pl.BlockSpec((RT, 1), lambda i: (i, 0)),
        ],
        out_shape=[
            jax.ShapeDtypeStruct((NP, H), jnp.float32),
            jax.ShapeDtypeStruct((NP, 1), jnp.float32),
        ],
    )(x_pad, Wx, bx2, degp)


def _tc_layer1_body(ragg_ref, dinv_ref, w_ref, b_ref, hp_ref):
    dv = dinv_ref[...]
    agg = (ragg_ref[0] + ragg_ref[1]) * dv
    h = jnp.maximum(jnp.dot(agg, w_ref[...], preferred_element_type=jnp.float32)
                    + b_ref[...], 0.0)
    hp_ref[...] = h * dv


def _tc_layer1(ragg, dinv, W, b2):
    grid = (NP // RT,)
    return pl.pallas_call(
        _tc_layer1_body,
        grid=grid,
        in_specs=[
            pl.BlockSpec((NC, RT, H), lambda i: (0, i, 0)),
            pl.BlockSpec((RT, 1), lambda i: (i, 0)),
            pl.BlockSpec((H, H), lambda i: (0, 0)),
            pl.BlockSpec((1, H), lambda i: (0, 0)),
        ],
        out_specs=pl.BlockSpec((RT, H), lambda i: (i, 0)),
        out_shape=jax.ShapeDtypeStruct((NP, H), jnp.float32),
    )(ragg, dinv, W, b2)


def _tc_layer2_body(ragg_ref, dinv_ref, w_ref, b_ref, a_ref, bb_ref, hA_ref, hB_ref):
    dv = dinv_ref[...]
    agg = (ragg_ref[0] + ragg_ref[1]) * dv
    t = jnp.maximum(jnp.dot(agg, w_ref[...], preferred_element_type=jnp.float32)
                    + b_ref[...], 0.0)
    hA_ref[...] = jnp.dot(t, a_ref[...], preferred_element_type=jnp.float32)
    hB_ref[...] = jnp.dot(t, bb_ref[...], preferred_element_type=jnp.float32)


def _tc_layer2(ragg, dinv, W, b2, A, B):
    grid = (NP // RT,)
    return pl.pallas_call(
        _tc_layer2_body,
        grid=grid,
        in_specs=[
            pl.BlockSpec((NC, RT, H), lambda i: (0, i, 0)),
            pl.BlockSpec((RT, 1), lambda i: (i, 0)),
            pl.BlockSpec((H, H), lambda i: (0, 0)),
            pl.BlockSpec((1, H), lambda i: (0, 0)),
            pl.BlockSpec((H, H), lambda i: (0, 0)),
            pl.BlockSpec((H, H), lambda i: (0, 0)),
        ],
        out_specs=[
            pl.BlockSpec((RT, H), lambda i: (i, 0)),
            pl.BlockSpec((RT, H), lambda i: (i, 0)),
        ],
        out_shape=[
            jax.ShapeDtypeStruct((NP, H), jnp.float32),
            jax.ShapeDtypeStruct((NP, H), jnp.float32),
        ],
    )(ragg, dinv, W, b2, A, B)


def kernel(x, edge_index, edge_attr, edge_weight, Wx, bx, We, be,
           Wg0, bg0, Wg1, bg1, Wd1, bd1, Wd2, bd2):
    src = edge_index[0]
    dst = edge_index[1]

    x_pad = jnp.pad(x, ((0, NP - N), (0, 0)))
    pad_idx = (jnp.arange(EP - E, dtype=jnp.int32) % (NP - N)) + N
    srcp = jnp.concatenate([src, pad_idx])
    dstp = jnp.concatenate([dst, pad_idx])
    ewp = jnp.pad(edge_weight, (0, EP - E))
    attrT = edge_attr.T

    A = Wd1[:H]
    B = Wd1[H:2 * H]
    C = Wd1[2 * H:]
    Wec = We @ C
    cvec = (be @ C + bd1).reshape(1, H)
    bx2 = bx.reshape(1, H)
    bg02 = bg0.reshape(1, H)
    bg12 = bg1.reshape(1, H)
    w2row = Wd2.reshape(1, H)
    b2s = bd2.reshape(1, 1)

    zeros_agg = jnp.zeros((NROWS_W, H), jnp.float32)

    degp = _sc_deg(dstp, ewp, zeros_agg)
    hp0, dinv = _tc_prep(x_pad, Wx, bx2, degp)

    ragg1 = _sc_gcn(hp0, srcp, dstp, ewp, zeros_agg)
    hp1 = _tc_layer1(ragg1, dinv, Wg0, bg02)
    ragg2 = _sc_gcn(hp1, srcp, dstp, ewp, zeros_agg)
    hA, hB = _tc_layer2(ragg2, dinv, Wg1, bg12, A, B)

    gab0 = _sc_gab(hA, hB, srcp, dstp, 0)
    gab1 = _sc_gab(hA, hB, srcp, dstp, 1)
    out0 = _tc_dec(gab0, attrT, Wec, cvec, w2row, b2s, EPH, 0)
    out1 = _tc_dec(gab1, attrT, Wec, cvec, w2row, b2s, E - EPH, EPH // RT)
    return jnp.concatenate([out0, out1], axis=0)


# speedup vs baseline: 1.9737x; 1.0855x over previous
import functools

import jax
import jax.numpy as jnp
from jax import lax
from jax.experimental import pallas as pl
from jax.experimental.pallas import tpu as pltpu
from jax.experimental.pallas import tpu_sc as plsc

N = 10000
NP = 10240
E = 320000
H = 128
DE = 16
NC, NS, L = 2, 16, 16
NW = NC * NS
CH = 128
KCH = 80
EPW = KCH * CH
EP = NW * EPW
RT = 512
NROWS_W = NP // NS

_mesh = lambda: plsc.VectorSubcoreMesh(core_axis_name="c", subcore_axis_name="s")


def _sc_deg_body(dstp, ewp, zeros_nb, out,
                 didx0, didx1, ewv0, ewv1, colbuf0, colbuf1, ssem0, ssem1, degS):
    c = lax.axis_index("c")
    s = lax.axis_index("s")
    wid = s * NC + c
    base = wid * EPW
    pltpu.sync_copy(zeros_nb, degS.at[pl.ds(s * NROWS_W, NROWS_W)])
    plsc.subcore_barrier()
    bufs = ((didx0, ewv0, colbuf0, ssem0), (didx1, ewv1, colbuf1, ssem1))

    def load(k, bi):
        didx, ewv, _, _ = bufs[bi]
        off = base + k * CH
        pltpu.sync_copy(dstp.at[pl.ds(off, CH)], didx)
        pltpu.sync_copy(ewp.at[pl.ds(off, CH)], ewv)

    def process(bi):
        didx, ewv, colbuf, ssem = bufs[bi]

        @plsc.parallel_loop(0, CH // 16, unroll=2)
        def _(t):
            ews = ewv[pl.ds(t * 16, 16)]
            for i in range(16):
                b = jnp.broadcast_to(ews[i], (16,))
                for j in range(H // 16):
                    colbuf[t * 16 + i, pl.ds(j * 16, 16)] = b

        return pltpu.async_copy(colbuf, degS.at[didx], ssem, add=True)

    load(0, 0)
    load(1, 1)

    def pair(k2, carry):
        k = 2 * k2
        cp0 = process(0)
        cp1 = process(1)
        cp0.wait()

        @pl.when(k2 < KCH // 2 - 1)
        def _():
            load(k + 2, 0)

        cp1.wait()

        @pl.when(k2 < KCH // 2 - 1)
        def _():
            load(k + 3, 1)

        return carry

    lax.fori_loop(0, KCH // 2, pair, 0)
    plsc.subcore_barrier()
    pltpu.sync_copy(degS.at[pl.ds(s * NROWS_W, NROWS_W)],
                    out.at[c, pl.ds(s * NROWS_W, NROWS_W)])


def _sc_deg(dstp, ewp, zeros_nb):
    k = pl.kernel(
        _sc_deg_body,
        out_type=jax.ShapeDtypeStruct((NC, NP, H), jnp.float32),
        mesh=_mesh(),
        compiler_params=pltpu.CompilerParams(needs_layout_passes=False),
        scratch_types=[
            pltpu.VMEM((CH,), jnp.int32),
            pltpu.VMEM((CH,), jnp.int32),
            pltpu.VMEM((CH,), jnp.float32),
            pltpu.VMEM((CH,), jnp.float32),
            pltpu.VMEM((CH, H), jnp.float32),
            pltpu.VMEM((CH, H), jnp.float32),
            pltpu.SemaphoreType.DMA,
            pltpu.SemaphoreType.DMA,
            pltpu.VMEM_SHARED((NP, H), jnp.float32),
        ],
    )
    return k(dstp, ewp, zeros_nb)


def _sc_gcn_body(hp, srcp, dstp, ewp, zeros_nb, out,
                 sidx0, sidx1, didx0, didx1, ewv0, ewv1, rows0, rows1,
                 gsem0, gsem1, ssem0, ssem1, aggS):
    c = lax.axis_index("c")
    s = lax.axis_index("s")
    wid = s * NC + c
    base = wid * EPW
    pltpu.sync_copy(zeros_nb, aggS.at[pl.ds(s * NROWS_W, NROWS_W)])
    plsc.subcore_barrier()
    bufs = ((sidx0, didx0, ewv0, rows0, gsem0, ssem0),
            (sidx1, didx1, ewv1, rows1, gsem1, ssem1))

    def load_and_gather(k, bi):
        sidx, didx, ewv, rows, gsem, _ = bufs[bi]
        off = base + k * CH
        pltpu.sync_copy(srcp.at[pl.ds(off, CH)], sidx)
        pltpu.async_copy(hp.at[sidx], rows, gsem)
        pltpu.sync_copy(dstp.at[pl.ds(off, CH)], didx)
        pltpu.sync_copy(ewp.at[pl.ds(off, CH)], ewv)

    def process(bi):
        sidx, didx, ewv, rows, gsem, ssem = bufs[bi]
        pltpu.make_async_copy(hp.at[sidx], rows, gsem).wait()

        @plsc.parallel_loop(0, CH // 16, unroll=2)
        def _(t):
            ews = ewv[pl.ds(t * 16, 16)]
            for i in range(16):
                sv = ews[i]
                e = t * 16 + i
                for j in range(H // 16):
                    rows[e, pl.ds(j * 16, 16)] = rows[e, pl.ds(j * 16, 16)] * sv

        return pltpu.async_copy(rows, aggS.at[didx], ssem, add=True)

    load_and_gather(0, 0)
    load_and_gather(1, 1)

    def pair(k2, carry):
        k = 2 * k2
        cp0 = process(0)
        cp1 = process(1)
        cp0.wait()

        @pl.when(k2 < KCH // 2 - 1)
        def _():
            load_and_gather(k + 2, 0)

        cp1.wait()

        @pl.when(k2 < KCH // 2 - 1)
        def _():
            load_and_gather(k + 3, 1)

        return carry

    lax.fori_loop(0, KCH // 2, pair, 0)
    plsc.subcore_barrier()
    pltpu.sync_copy(aggS.at[pl.ds(s * NROWS_W, NROWS_W)],
                    out.at[c, pl.ds(s * NROWS_W, NROWS_W)])


def _sc_gcn(hp, srcp, dstp, ewp, zeros_nb):
    k = pl.kernel(
        _sc_gcn_body,
        out_type=jax.ShapeDtypeStruct((NC, NP, H), jnp.float32),
        mesh=_mesh(),
        compiler_params=pltpu.CompilerParams(needs_layout_passes=False),
        scratch_types=[
            pltpu.VMEM((CH,), jnp.int32),
            pltpu.VMEM((CH,), jnp.int32),
            pltpu.VMEM((CH,), jnp.int32),
            pltpu.VMEM((CH,), jnp.int32),
            pltpu.VMEM((CH,), jnp.float32),
            pltpu.VMEM((CH,), jnp.float32),
            pltpu.VMEM((CH, H), jnp.float32),
            pltpu.VMEM((CH, H), jnp.float32),
            pltpu.SemaphoreType.DMA,
            pltpu.SemaphoreType.DMA,
            pltpu.SemaphoreType.DMA,
            pltpu.SemaphoreType.DMA,
            pltpu.VMEM_SHARED((NP, H), jnp.float32),
        ],
    )
    return k(hp, srcp, dstp, ewp, zeros_nb)


EPH = EP // 2
EPWH = EPH // NW
KCHH = EPWH // CH


def _sc_gab_body(half, hA, hB, srcp, dstp, out,
                 aidx0, aidx1, bidx0, bidx1, bufA0, bufA1, bufB0, bufB1,
                 semA0, semA1, semB0, semB1, wsem0, wsem1):
    c = lax.axis_index("c")
    s = lax.axis_index("s")
    wid = s * NC + c
    base = half * EPH + wid * EPWH
    obase = wid * EPWH
    bufs = ((aidx0, bidx0, bufA0, bufB0, semA0, semB0, wsem0),
            (aidx1, bidx1, bufA1, bufB1, semA1, semB1, wsem1))

    def load_and_gather(k, bi):
        aidx, bidx, bufA, bufB, semA, semB, _ = bufs[bi]
        off = base + k * CH
        pltpu.sync_copy(srcp.at[pl.ds(off, CH)], aidx)
        pltpu.sync_copy(dstp.at[pl.ds(off, CH)], bidx)
        pltpu.async_copy(hA.at[aidx], bufA, semA)
        pltpu.async_copy(hB.at[bidx], bufB, semB)

    def process(k, bi):
        aidx, bidx, bufA, bufB, semA, semB, wsem = bufs[bi]
        ooff = obase + k * CH
        pltpu.make_async_copy(hA.at[aidx], bufA, semA).wait()
        pltpu.make_async_copy(hB.at[bidx], bufB, semB).wait()

        @plsc.parallel_loop(0, CH // 16, unroll=2)
        def _(t):
            for i in range(16):
                e = t * 16 + i
                for j in range(H // 16):
                    sl = pl.ds(j * 16, 16)
                    bufA[e, sl] = bufA[e, sl] + bufB[e, sl]

        return pltpu.async_copy(bufA, out.at[pl.ds(ooff, CH)], wsem)

    load_and_gather(0, 0)
    load_and_gather(1, 1)

    def pair(k2, carry):
        k = 2 * k2
        cp0 = process(k, 0)
        cp1 = process(k + 1, 1)
        cp0.wait()

        @pl.when(k2 < KCHH // 2 - 1)
        def _():
            load_and_gather(k + 2, 0)

        cp1.wait()

        @pl.when(k2 < KCHH // 2 - 1)
        def _():
            load_and_gather(k + 3, 1)

        return carry

    lax.fori_loop(0, KCHH // 2, pair, 0)


def _sc_gab(hA, hB, srcp, dstp, half):
    k = pl.kernel(
        functools.partial(_sc_gab_body, half),
        out_type=jax.ShapeDtypeStruct((EPH, H), jnp.float32),
        mesh=_mesh(),
        compiler_params=pltpu.CompilerParams(needs_layout_passes=False),
        scratch_types=[
            pltpu.VMEM((CH,), jnp.int32),
            pltpu.VMEM((CH,), jnp.int32),
            pltpu.VMEM((CH,), jnp.int32),
            pltpu.VMEM((CH,), jnp.int32),
            pltpu.VMEM((CH, H), jnp.float32),
            pltpu.VMEM((CH, H), jnp.float32),
            pltpu.VMEM((CH, H), jnp.float32),
            pltpu.VMEM((CH, H), jnp.float32),
            pltpu.SemaphoreType.DMA,
            pltpu.SemaphoreType.DMA,
            pltpu.SemaphoreType.DMA,
            pltpu.SemaphoreType.DMA,
            pltpu.SemaphoreType.DMA,
            pltpu.SemaphoreType.DMA,
        ],
    )
    return k(hA, hB, srcp, dstp)


def _tc_dec_body(gab_ref, attrT_ref, wec_ref, cvec_ref, w2_ref, b2_ref, out_ref):
    ea = lax.dot_general(attrT_ref[...], wec_ref[...],
                         (((0,), (0,)), ((), ())),
                         preferred_element_type=jnp.float32)
    t = jnp.maximum(gab_ref[...] + ea + cvec_ref[...], 0.0)
    out_ref[...] = jnp.sum(t * w2_ref[...], axis=1, keepdims=True) + b2_ref[...]


DECRT = 1024


def _tc_dec(gab, attrT, Wec, cvec2, w2row, b2s, n_rows, attr_off):
    grid = (n_rows // DECRT,)
    return pl.pallas_call(
        _tc_dec_body,
        grid=grid,
        in_specs=[
            pl.BlockSpec((DECRT, H), lambda i: (i, 0)),
            pl.BlockSpec((DE, DECRT), lambda i: (0, i + attr_off)),
            pl.BlockSpec((DE, H), lambda i: (0, 0)),
            pl.BlockSpec((1, H), lambda i: (0, 0)),
            pl.BlockSpec((1, H), lambda i: (0, 0)),
            pl.BlockSpec((1, 1), lambda i: (0, 0)),
        ],
        out_specs=pl.BlockSpec((DECRT, 1), lambda i: (i, 0)),
        out_shape=jax.ShapeDtypeStruct((n_rows, 1), jnp.float32),
    )(gab, attrT, Wec, cvec2, w2row, b2s)


def _tc_prep_body(x_ref, wx_ref, bx_ref, degp_ref, hp0_ref, dinv_ref):
    d = degp_ref[0][:, 0:1] + degp_ref[1][:, 0:1]
    dinv = jnp.where(d > 0, lax.rsqrt(jnp.maximum(d, 1e-12)), 0.0)
    h = jnp.dot(x_ref[...], wx_ref[...], preferred_element_type=jnp.float32) + bx_ref[...]
    hp0_ref[...] = h * dinv
    dinv_ref[...] = dinv


def _tc_prep(x_pad, Wx, bx2, degp):
    grid = (NP // RT,)
    return pl.pallas_call(
        _tc_prep_body,
        grid=grid,
        in_specs=[
            pl.BlockSpec((RT, H), lambda i: (i, 0)),
            pl.BlockSpec((H, H), lambda i: (0, 0)),
            pl.BlockSpec((1, H), lambda i: (0, 0)),
            pl.BlockSpec((NC, RT, H), lambda i: (0, i, 0)),
        ],
        out_specs=[
            pl.BlockSpec((RT, H), lambda i: (i, 0)),
            pl.BlockSpec((RT, 1), lambda i: (i, 0)),
        ],
        out_shape=[
            jax.ShapeDtypeStruct((NP, H), jnp.float32),
            jax.ShapeDtypeStruct((NP, 1), jnp.float32),
        ],
    )(x_pad, Wx, bx2, degp)


def _tc_layer1_body(ragg_ref, dinv_ref, w_ref, b_ref, hp_ref):
    dv = dinv_ref[...]
    agg = (ragg_ref[0] + ragg_ref[1]) * dv
    h = jnp.maximum(jnp.dot(agg, w_ref[...], preferred_element_type=jnp.float32)
                    + b_ref[...], 0.0)
    hp_ref[...] = h * dv


def _tc_layer1(ragg, dinv, W, b2):
    grid = (NP // RT,)
    return pl.pallas_call(
        _tc_layer1_body,
        grid=grid,
        in_specs=[
            pl.BlockSpec((NC, RT, H), lambda i: (0, i, 0)),
            pl.BlockSpec((RT, 1), lambda i: (i, 0)),
            pl.BlockSpec((H, H), lambda i: (0, 0)),
            pl.BlockSpec((1, H), lambda i: (0, 0)),
        ],
        out_specs=pl.BlockSpec((RT, H), lambda i: (i, 0)),
        out_shape=jax.ShapeDtypeStruct((NP, H), jnp.float32),
    )(ragg, dinv, W, b2)


def _tc_layer2_body(ragg_ref, dinv_ref, w_ref, b_ref, a_ref, bb_ref, hA_ref, hB_ref):
    dv = dinv_ref[...]
    agg = (ragg_ref[0] + ragg_ref[1]) * dv
    t = jnp.maximum(jnp.dot(agg, w_ref[...], preferred_element_type=jnp.float32)
                    + b_ref[...], 0.0)
    hA_ref[...] = jnp.dot(t, a_ref[...], preferred_element_type=jnp.float32)
    hB_ref[...] = jnp.dot(t, bb_ref[...], preferred_element_type=jnp.float32)


def _tc_layer2(ragg, dinv, W, b2, A, B):
    grid = (NP // RT,)
    return pl.pallas_call(
        _tc_layer2_body,
        grid=grid,
        in_specs=[
            pl.BlockSpec((NC, RT, H), lambda i: (0, i, 0)),
            pl.BlockSpec((RT, 1), lambda i: (i, 0)),
            pl.BlockSpec((H, H), lambda i: (0, 0)),
            pl.BlockSpec((1, H), lambda i: (0, 0)),
            pl.BlockSpec((H, H), lambda i: (0, 0)),
            pl.BlockSpec((H, H), lambda i: (0, 0)),
        ],
        out_specs=[
            pl.BlockSpec((RT, H), lambda i: (i, 0)),
            pl.BlockSpec((RT, H), lambda i: (i, 0)),
        ],
        out_shape=[
            jax.ShapeDtypeStruct((NP, H), jnp.float32),
            jax.ShapeDtypeStruct((NP, H), jnp.float32),
        ],
    )(ragg, dinv, W, b2, A, B)


def kernel(x, edge_index, edge_attr, edge_weight, Wx, bx, We, be,
           Wg0, bg0, Wg1, bg1, Wd1, bd1, Wd2, bd2):
    src = edge_index[0]
    dst = edge_index[1]

    x_pad = jnp.pad(x, ((0, NP - N), (0, 0)))
    pad_idx = (jnp.arange(EP - E, dtype=jnp.int32) % (NP - N)) + N
    srcp = jnp.concatenate([src, pad_idx])
    dstp = jnp.concatenate([dst, pad_idx])
    ewp = jnp.pad(edge_weight, (0, EP - E))
    attrT = edge_attr.T
    attrTP = jnp.pad(attrT, ((0, 0), (0, EP - E)))

    A = Wd1[:H]
    B = Wd1[H:2 * H]
    C = Wd1[2 * H:]
    Wec = We @ C
    cvec = (be @ C + bd1).reshape(1, H)
    bx2 = bx.reshape(1, H)
    bg02 = bg0.reshape(1, H)
    bg12 = bg1.reshape(1, H)
    w2row = Wd2.reshape(1, H)
    b2s = bd2.reshape(1, 1)

    zeros_agg = jnp.zeros((NROWS_W, H), jnp.float32)

    degp = _sc_deg(dstp, ewp, zeros_agg)
    hp0, dinv = _tc_prep(x_pad, Wx, bx2, degp)

    ragg1 = _sc_gcn(hp0, srcp, dstp, ewp, zeros_agg)
    hp1 = _tc_layer1(ragg1, dinv, Wg0, bg02)
    ragg2 = _sc_gcn(hp1, srcp, dstp, ewp, zeros_agg)
    hA, hB = _tc_layer2(ragg2, dinv, Wg1, bg12, A, B)

    gab0 = _sc_gab(hA, hB, srcp, dstp, 0)
    gab1 = _sc_gab(hA, hB, srcp, dstp, 1)
    n2 = -(-(E - EPH) // DECRT) * DECRT
    out0 = _tc_dec(gab0, attrT, Wec, cvec, w2row, b2s, EPH, 0)
    out1 = _tc_dec(gab1, attrTP, Wec, cvec, w2row, b2s, n2, EPH // DECRT)
    return jnp.concatenate([out0, out1[:E - EPH]], axis=0)
